# Initial kernel scaffold; baseline (speedup 1.0000x reference)
import jax
import jax.numpy as jnp
from jax.experimental import pallas as pl


def kernel(patches):
    raise NotImplementedError("write your pallas kernel here")


# speedup vs baseline: 37.8420x; 37.8420x over previous
import functools
import math
import random

import numpy as np
import jax
import jax.numpy as jnp
from jax import lax
from jax.experimental import pallas as pl
from jax.experimental.pallas import tpu as pltpu
from jax.experimental.pallas import tpu_sc as plsc


@functools.lru_cache(maxsize=None)
def _shuffle_tables(T, B):
    n = int(math.sqrt(T))

    def one(rand):
        fi = np.arange(T).reshape(n, n)
        if rand == 0:
            a = fi[:, 0::2].copy()
            b = fi[:, 1::2].copy()
        else:
            a = fi[:, 1::2].copy()
            b = fi[:, 0::2].copy()
        for i in range(16):
            if i % 2 != 0:
                tmp = a[i].copy()
                a[i] = b[i]
                b[i] = tmp
        fwd = np.concatenate((a, b)).reshape(-1)
        return fwd, np.argsort(fwd)

    rng = random.Random(0)
    pairs = [one(rng.randint(0, 1)) for _ in range(B)]
    fwd = np.stack([p[0] for p in pairs], axis=-1).astype(np.int32)
    bwd = np.stack([p[1] for p in pairs], axis=-1).astype(np.int32)
    return fwd, bwd


@functools.lru_cache(maxsize=None)
def _build_gather(R, C, n_rows):
    info = plsc.get_sparse_core_info()
    num_workers = info.num_cores * info.num_subcores
    rows_w = R // num_workers
    chunk = 64
    n_chunks = rows_w // chunk
    assert rows_w % chunk == 0

    mesh = plsc.VectorSubcoreMesh(core_axis_name="c", subcore_axis_name="s")

    @functools.partial(
        pl.kernel,
        mesh=mesh,
        out_type=jax.ShapeDtypeStruct((R, C), jnp.float32),
        scratch_types=[
            pltpu.VMEM((rows_w,), jnp.int32),
            pltpu.VMEM((chunk, C), jnp.float32),
            pltpu.VMEM((chunk, C), jnp.float32),
            pltpu.SemaphoreType.DMA,
            pltpu.SemaphoreType.DMA,
            pltpu.SemaphoreType.DMA,
            pltpu.SemaphoreType.DMA,
        ],
    )
    def gather_rows(flat_hbm, gidx_hbm, out_hbm, idx_v, buf0, buf1,
                    g0, g1, s0, s1):
        wid = lax.axis_index("s") * info.num_cores + lax.axis_index("c")
        base = wid * rows_w
        pltpu.sync_copy(gidx_hbm.at[pl.ds(base, rows_w)], idx_v)
        bufs = (buf0, buf1)
        gsem = (g0, g1)
        ssem = (s0, s1)
        gathers = [None] * n_chunks
        scatters = [None] * n_chunks
        gathers[0] = pltpu.async_copy(
            flat_hbm.at[idx_v.at[pl.ds(0, chunk)]], bufs[0], gsem[0])
        for c in range(n_chunks):
            b = c % 2
            gathers[c].wait()
            if c + 1 < n_chunks:
                if c >= 1:
                    scatters[c - 1].wait()
                gathers[c + 1] = pltpu.async_copy(
                    flat_hbm.at[idx_v.at[pl.ds((c + 1) * chunk, chunk)]],
                    bufs[1 - b], gsem[1 - b])
            scatters[c] = pltpu.async_copy(
                bufs[b], out_hbm.at[pl.ds(base + c * chunk, chunk)], ssem[b])
        scatters[n_chunks - 2].wait()
        scatters[n_chunks - 1].wait()

    return gather_rows


def kernel(patches):
    T, B, C = patches.shape
    remain_T = T // 2
    fwd_np, bwd_np = _shuffle_tables(T, B)
    gidx_np = (fwd_np[:remain_T].astype(np.int64) * B
               + np.arange(B, dtype=np.int64)[None, :]).reshape(-1)
    gidx = jnp.asarray(gidx_np.astype(np.int32))
    flat = patches.reshape(T * B, C)
    R = remain_T * B
    out = _build_gather(R, C, T * B)(flat, gidx)
    shuffled = out.reshape(remain_T, B, C)
    return (shuffled,
            jnp.asarray(fwd_np, dtype=jnp.int32),
            jnp.asarray(bwd_np, dtype=jnp.int32))


# speedup vs baseline: 38.1185x; 1.0073x over previous
import functools
import math
import random

import numpy as np
import jax
import jax.numpy as jnp
from jax import lax
from jax.experimental import pallas as pl
from jax.experimental.pallas import tpu as pltpu
from jax.experimental.pallas import tpu_sc as plsc


@functools.lru_cache(maxsize=None)
def _shuffle_tables(T, B):
    n = int(math.sqrt(T))

    def one(rand):
        fi = np.arange(T).reshape(n, n)
        if rand == 0:
            a = fi[:, 0::2].copy()
            b = fi[:, 1::2].copy()
        else:
            a = fi[:, 1::2].copy()
            b = fi[:, 0::2].copy()
        for i in range(16):
            if i % 2 != 0:
                tmp = a[i].copy()
                a[i] = b[i]
                b[i] = tmp
        fwd = np.concatenate((a, b)).reshape(-1)
        return fwd, np.argsort(fwd)

    rng = random.Random(0)
    pairs = [one(rng.randint(0, 1)) for _ in range(B)]
    fwd = np.stack([p[0] for p in pairs], axis=-1).astype(np.int32)
    bwd = np.stack([p[1] for p in pairs], axis=-1).astype(np.int32)
    return fwd, bwd


@functools.lru_cache(maxsize=None)
def _build_gather(R, C, n_rows):
    info = plsc.get_sparse_core_info()
    num_workers = info.num_cores * info.num_subcores
    rows_w = R // num_workers
    chunk = 32
    nbuf = 4
    n_chunks = rows_w // chunk
    assert rows_w % chunk == 0 and n_chunks >= nbuf

    mesh = plsc.VectorSubcoreMesh(core_axis_name="c", subcore_axis_name="s")

    @functools.partial(
        pl.kernel,
        mesh=mesh,
        out_type=jax.ShapeDtypeStruct((R, C), jnp.float32),
        scratch_types=(
            [pltpu.VMEM((rows_w,), jnp.int32)]
            + [pltpu.VMEM((chunk, C), jnp.float32) for _ in range(nbuf)]
            + [pltpu.SemaphoreType.DMA for _ in range(2 * nbuf)]
        ),
    )
    def gather_rows(flat_hbm, gidx_hbm, out_hbm, idx_v, *bufs_and_sems):
        bufs = bufs_and_sems[:nbuf]
        gsem = bufs_and_sems[nbuf:2 * nbuf]
        ssem = bufs_and_sems[2 * nbuf:]
        wid = lax.axis_index("s") * info.num_cores + lax.axis_index("c")
        base = wid * rows_w
        pltpu.sync_copy(gidx_hbm.at[pl.ds(base, rows_w)], idx_v)

        def gather(c):
            b = c % nbuf
            return pltpu.async_copy(
                flat_hbm.at[idx_v.at[pl.ds(c * chunk, chunk)]],
                bufs[b], gsem[b])

        gathers = [None] * n_chunks
        scatters = [None] * n_chunks
        for c in range(nbuf - 1):
            gathers[c] = gather(c)
        for c in range(n_chunks):
            b = c % nbuf
            gathers[c].wait()
            nxt = c + nbuf - 1
            if nxt < n_chunks:
                if c >= 1:
                    scatters[c - 1].wait()
                gathers[nxt] = gather(nxt)
            scatters[c] = pltpu.async_copy(
                bufs[b], out_hbm.at[pl.ds(base + c * chunk, chunk)], ssem[b])
        for c in range(max(0, n_chunks - nbuf), n_chunks):
            scatters[c].wait()

    return gather_rows


def kernel(patches):
    T, B, C = patches.shape
    remain_T = T // 2
    fwd_np, bwd_np = _shuffle_tables(T, B)
    gidx_np = (fwd_np[:remain_T].astype(np.int64) * B
               + np.arange(B, dtype=np.int64)[None, :]).reshape(-1)
    gidx = jnp.asarray(gidx_np.astype(np.int32))
    flat = patches.reshape(T * B, C)
    R = remain_T * B
    out = _build_gather(R, C, T * B)(flat, gidx)
    shuffled = out.reshape(remain_T, B, C)
    return (shuffled,
            jnp.asarray(fwd_np, dtype=jnp.int32),
            jnp.asarray(bwd_np, dtype=jnp.int32))


# speedup vs baseline: 39.3600x; 1.0326x over previous
import functools
import math
import random

import numpy as np
import jax
import jax.numpy as jnp
from jax import lax
from jax.experimental import pallas as pl
from jax.experimental.pallas import tpu as pltpu
from jax.experimental.pallas import tpu_sc as plsc


@functools.lru_cache(maxsize=None)
def _shuffle_tables(T, B):
    n = int(math.sqrt(T))

    def one(rand):
        fi = np.arange(T).reshape(n, n)
        if rand == 0:
            a = fi[:, 0::2].copy()
            b = fi[:, 1::2].copy()
        else:
            a = fi[:, 1::2].copy()
            b = fi[:, 0::2].copy()
        for i in range(16):
            if i % 2 != 0:
                tmp = a[i].copy()
                a[i] = b[i]
                b[i] = tmp
        fwd = np.concatenate((a, b)).reshape(-1)
        return fwd, np.argsort(fwd)

    rng = random.Random(0)
    pairs = [one(rng.randint(0, 1)) for _ in range(B)]
    fwd = np.stack([p[0] for p in pairs], axis=-1).astype(np.int32)
    bwd = np.stack([p[1] for p in pairs], axis=-1).astype(np.int32)
    return fwd, bwd


@functools.lru_cache(maxsize=None)
def _build_gather(R, C, n_rows):
    info = plsc.get_sparse_core_info()
    num_workers = info.num_cores * info.num_subcores
    rows_w = R // num_workers
    chunk = 32
    nbuf = 4
    n_chunks = rows_w // chunk
    assert rows_w % chunk == 0 and n_chunks >= nbuf

    mesh = plsc.VectorSubcoreMesh(core_axis_name="c", subcore_axis_name="s")

    @functools.partial(
        pl.kernel,
        mesh=mesh,
        out_type=jax.ShapeDtypeStruct((R, C), jnp.float32),
        scratch_types=(
            [pltpu.VMEM((rows_w,), jnp.int32)]
            + [pltpu.VMEM((chunk, C), jnp.float32) for _ in range(nbuf)]
            + [pltpu.SemaphoreType.DMA for _ in range(2 * nbuf)]
        ),
    )
    def gather_rows(flat_hbm, gidx_hbm, out_hbm, idx_v, *bufs_and_sems):
        bufs = bufs_and_sems[:nbuf]
        gsem = bufs_and_sems[nbuf:2 * nbuf]
        ssem = bufs_and_sems[2 * nbuf:]
        wid = lax.axis_index("s") * info.num_cores + lax.axis_index("c")
        base = wid * rows_w
        pltpu.sync_copy(gidx_hbm.at[pl.ds(base, rows_w)], idx_v)

        def gather_desc(c, j):
            return pltpu.make_async_copy(
                flat_hbm.at[idx_v.at[pl.ds(c * chunk, chunk)]],
                bufs[j], gsem[j])

        def scatter_desc(c, j):
            return pltpu.make_async_copy(
                bufs[j], out_hbm.at[pl.ds(base + c * chunk, chunk)], ssem[j])

        n_groups = n_chunks // nbuf
        assert n_chunks % nbuf == 0 and n_groups >= 2

        for j in range(nbuf):
            gather_desc(j, j).start()
        for j in range(nbuf):
            gather_desc(j, j).wait()
            scatter_desc(j, j).start()
            gather_desc(nbuf + j, j).start()

        def group_body(g, carry):
            c0 = g * nbuf
            for j in range(nbuf):
                c = c0 + j
                scatter_desc(c - nbuf, j).wait()
                gather_desc(c, j).wait()
                scatter_desc(c, j).start()
                gather_desc(c + nbuf, j).start()
            return carry

        if n_groups > 2:
            lax.fori_loop(1, n_groups - 1, group_body, 0, unroll=False)

        c0 = (n_groups - 1) * nbuf
        for j in range(nbuf):
            scatter_desc(c0 + j - nbuf, j).wait()
            gather_desc(c0 + j, j).wait()
            scatter_desc(c0 + j, j).start()
        for j in range(nbuf):
            scatter_desc(c0 + j, j).wait()

    return gather_rows


def kernel(patches):
    T, B, C = patches.shape
    remain_T = T // 2
    fwd_np, bwd_np = _shuffle_tables(T, B)
    gidx_np = (fwd_np[:remain_T].astype(np.int64) * B
               + np.arange(B, dtype=np.int64)[None, :]).reshape(-1)
    gidx = jnp.asarray(gidx_np.astype(np.int32))
    flat = patches.reshape(T * B, C)
    R = remain_T * B
    out = _build_gather(R, C, T * B)(flat, gidx)
    shuffled = out.reshape(remain_T, B, C)
    return (shuffled,
            jnp.asarray(fwd_np, dtype=jnp.int32),
            jnp.asarray(bwd_np, dtype=jnp.int32))
